# Initial kernel scaffold; baseline (speedup 1.0000x reference)
import jax
import jax.numpy as jnp
from jax.experimental import pallas as pl


def kernel(timestamps, session_starts, abs_tab, rel_tab, sess_tab, W, b):
    raise NotImplementedError("write your pallas kernel here")


# speedup vs baseline: 6.2710x; 6.2710x over previous
import functools
import math

import jax
import jax.numpy as jnp
import numpy as np
from jax.experimental import pallas as pl
from jax.experimental.pallas import tpu as pltpu

_NUM_BUCKETS = 100
_EMBED_DIM = 128
_MAX_VAL = 1000000.0
_I32_MAX = np.int32(2**31 - 1)
_I32_MIN = np.int32(-(2**31))

_BASES = np.round(
    np.exp(np.arange(_NUM_BUCKETS) * (math.log(_MAX_VAL) / (_NUM_BUCKETS - 1)))
).astype(np.int64)
_CANDS = (_BASES[:, None] + np.arange(-16, 16)[None, :]).astype(np.int32)


def _quantize_f32(t_i32):
    clamped = jnp.clip(t_i32.astype(jnp.float32), 1.0, None)
    log_times = jnp.log(clamped) / math.log(_MAX_VAL) * (_NUM_BUCKETS - 1)
    return jnp.clip(log_times.astype(jnp.int32), 0, _NUM_BUCKETS - 1)


def _bucket_bounds(zero):
    del zero
    cands = jax.lax.optimization_barrier(jnp.asarray(_CANDS))
    q = _quantize_f32(cands)
    ok = q >= jnp.arange(_NUM_BUCKETS, dtype=jnp.int32)[:, None]
    th = jnp.min(jnp.where(ok, cands, _I32_MAX), axis=1).astype(jnp.int32)
    lo = th.at[0].set(_I32_MIN)
    hi = jnp.concatenate([th[1:], jnp.full((1,), _I32_MAX, jnp.int32)])
    pad = jnp.full((_EMBED_DIM - _NUM_BUCKETS,), _I32_MAX, jnp.int32)
    lo = jnp.concatenate([lo, pad]).reshape(1, _EMBED_DIM)
    hi = jnp.concatenate([hi, pad]).reshape(1, _EMBED_DIM)
    return lo, hi


def _project_tables_kernel(tabs_ref, w3_ref, b_ref, out_ref):
    for p in range(3):
        acc = jax.lax.dot_general(
            tabs_ref[p], w3_ref[p], (((1,), (1,)), ((), ())),
            preferred_element_type=jnp.float32)
        if p == 0:
            acc = acc + b_ref[:]
        out_ref[p] = acc


def _encode_kernel(ta_ref, tr_ref, ts_ref, lo_ref, hi_ref, p_ref, out_ref):
    lo = lo_ref[:]
    hi = hi_ref[:]

    def onehot(t_col):
        return jnp.logical_and(t_col >= lo, t_col < hi).astype(jnp.float32)

    acc = jax.lax.dot_general(
        onehot(ta_ref[:]), p_ref[0], (((1,), (0,)), ((), ())),
        preferred_element_type=jnp.float32)
    acc += jax.lax.dot_general(
        onehot(tr_ref[:]), p_ref[1], (((1,), (0,)), ((), ())),
        preferred_element_type=jnp.float32)
    acc += jax.lax.dot_general(
        onehot(ts_ref[:]), p_ref[2], (((1,), (0,)), ((), ())),
        preferred_element_type=jnp.float32)
    out_ref[:] = acc


def kernel(timestamps, session_starts, abs_tab, rel_tab, sess_tab, W, b):
    B, L = timestamps.shape
    T = B * L
    d3 = _EMBED_DIM // 3

    ts = timestamps.astype(jnp.int32)
    t_rel = jnp.concatenate(
        [jnp.zeros((B, 1), jnp.int32), ts[:, 1:] - ts[:, :-1]], axis=1)
    t_sess = ts - session_starts.astype(jnp.int32)[:, None]

    lo, hi = _bucket_bounds(ts[0, 0] * 0)

    def pad_tab(t):
        return jnp.pad(t, ((0, _EMBED_DIM - _NUM_BUCKETS), (0, 48 - t.shape[1])))

    tabs = jnp.stack([pad_tab(abs_tab), pad_tab(rel_tab), pad_tab(sess_tab)])
    w3 = jnp.stack([
        jnp.pad(W[:, 0:d3], ((0, 0), (0, 6))),
        jnp.pad(W[:, d3:2 * d3], ((0, 0), (0, 6))),
        jnp.pad(W[:, 2 * d3:], ((0, 0), (0, 4))),
    ])

    proj = pl.pallas_call(
        _project_tables_kernel,
        out_shape=jax.ShapeDtypeStruct((3, _EMBED_DIM, _EMBED_DIM), jnp.float32),
        in_specs=[
            pl.BlockSpec((3, _EMBED_DIM, 48), lambda: (0, 0, 0)),
            pl.BlockSpec((3, _EMBED_DIM, 48), lambda: (0, 0, 0)),
            pl.BlockSpec((1, _EMBED_DIM), lambda: (0, 0)),
        ],
        out_specs=pl.BlockSpec((3, _EMBED_DIM, _EMBED_DIM), lambda: (0, 0, 0)),
    )(tabs, w3, b.reshape(1, _EMBED_DIM))

    BM = 1024
    grid = (T // BM,)
    out = pl.pallas_call(
        _encode_kernel,
        grid=grid,
        out_shape=jax.ShapeDtypeStruct((T, _EMBED_DIM), jnp.float32),
        in_specs=[
            pl.BlockSpec((BM, 1), lambda i: (i, 0)),
            pl.BlockSpec((BM, 1), lambda i: (i, 0)),
            pl.BlockSpec((BM, 1), lambda i: (i, 0)),
            pl.BlockSpec((1, _EMBED_DIM), lambda i: (0, 0)),
            pl.BlockSpec((1, _EMBED_DIM), lambda i: (0, 0)),
            pl.BlockSpec((3, _EMBED_DIM, _EMBED_DIM), lambda i: (0, 0, 0)),
        ],
        out_specs=pl.BlockSpec((BM, _EMBED_DIM), lambda i: (i, 0)),
        compiler_params=pltpu.CompilerParams(
            dimension_semantics=("arbitrary",)),
    )(ts.reshape(T, 1), t_rel.reshape(T, 1), t_sess.reshape(T, 1), lo, hi, proj)

    return out.reshape(B, L, _EMBED_DIM)
